# baseline (device time: 194454 ns/iter reference)
import jax
import jax.numpy as jnp
from jax import lax
from jax.experimental import pallas as pl
from jax.experimental.pallas import tpu as pltpu

N_DEV = 4
SQ = 1024
SKV_PER = 1024
HQ = 32
HQ_PER = 8
DH = 128
DM = 1024
HD_PER = HQ_PER * DH
HD_ALL = HQ * DH
SCALE = 0.08838834764831843
LOCAL_W = 128
N_GLOB = 32
LOC_TILE = 248
W_WIN = 512
CH1_ROWS = 136
KV_EXT = SKV_PER + CH1_ROWS
QROWS = SQ // N_DEV


def kernel(x, Wq, K_ext, V_ext, Wo):
    xb = x.reshape(SQ, DM).astype(jnp.bfloat16)
    Wqb = Wq.astype(jnp.bfloat16)
    Wob = Wo.astype(jnp.bfloat16)
    K2 = K_ext.reshape(SKV_PER, HD_ALL).astype(jnp.bfloat16)
    V2 = V_ext.reshape(SKV_PER, HD_ALL).astype(jnp.bfloat16)

    def body(x_ref, wq_ref, k_any, v_any, wo_ref, out_ref,
             kall, vall, kst, vst, qbuf, ctx_buf, qg_buf,
             ctxT, mlT, gctxT, gml, rs_buf,
             k_send, v_send, k_recv, v_recv,
             qg_send, qg_recv, gc_send, gc_recv, gm_send, gm_recv,
             rs_send, rs_recv, ag_send, ag_recv, st_sem):
        my = lax.axis_index("i")

        barrier_sem = pltpu.get_barrier_semaphore()
        for d in range(1, N_DEV):
            peer = lax.rem(my + d, N_DEV)
            pl.semaphore_signal(
                barrier_sem, inc=1,
                device_id=(peer,), device_id_type=pl.DeviceIdType.MESH,
            )
        pl.semaphore_wait(barrier_sem, N_DEV - 1)

        @pl.when(my == 0)
        def _():
            for t in range(N_DEV):
                for (src, dst, ssem, rsem) in (
                    (k_any, kall, k_send, k_recv),
                    (v_any, vall, v_send, v_recv),
                ):
                    pltpu.make_async_remote_copy(
                        src_ref=src.at[:, pl.ds(t * HD_PER, HD_PER)],
                        dst_ref=dst.at[pl.ds(0, SKV_PER), :],
                        send_sem=ssem.at[t],
                        recv_sem=rsem.at[0],
                        device_id=(t,),
                        device_id_type=pl.DeviceIdType.MESH,
                    ).start()

        @pl.when(my == 1)
        def _():
            for t in range(N_DEV):
                for (src, dst, ssem, rsem) in (
                    (k_any, kall, k_send, k_recv),
                    (v_any, vall, v_send, v_recv),
                ):
                    pltpu.make_async_remote_copy(
                        src_ref=src.at[pl.ds(0, CH1_ROWS),
                                       pl.ds(t * HD_PER, HD_PER)],
                        dst_ref=dst.at[pl.ds(SKV_PER, CH1_ROWS), :],
                        send_sem=ssem.at[t],
                        recv_sem=rsem.at[1],
                        device_id=(t,),
                        device_id_type=pl.DeviceIdType.MESH,
                    ).start()

        qg32 = jnp.dot(x_ref[:N_GLOB, :], wq_ref[:, :],
                       preferred_element_type=jnp.float32)
        qg_buf[:, pl.ds(my * HD_PER, HD_PER)] = qg32.astype(jnp.bfloat16)
        qg_sends = []
        for d in range(1, N_DEV):
            t = lax.rem(my + d, N_DEV)
            rdma = pltpu.make_async_remote_copy(
                src_ref=qg_buf.at[:, pl.ds(my * HD_PER, HD_PER)],
                dst_ref=qg_buf.at[:, pl.ds(my * HD_PER, HD_PER)],
                send_sem=qg_send.at[d - 1],
                recv_sem=qg_recv.at[d - 1],
                device_id=(t,),
                device_id_type=pl.DeviceIdType.MESH,
            )
            rdma.start()
            qg_sends.append(rdma)

        def start_stage(grp, slot):
            s0 = slot * SKV_PER
            dk = pltpu.make_async_copy(
                k_any.at[:, pl.ds(grp * HD_PER, HD_PER)],
                kst.at[pl.ds(s0, SKV_PER), :], st_sem.at[2 * slot])
            dv = pltpu.make_async_copy(
                v_any.at[:, pl.ds(grp * HD_PER, HD_PER)],
                vst.at[pl.ds(s0, SKV_PER), :], st_sem.at[2 * slot + 1])
            dk.start()
            dv.start()
            return (dk, dv)

        pend = start_stage(0, 0)

        q32 = jnp.dot(x_ref[:, :], wq_ref[:, :],
                      preferred_element_type=jnp.float32)
        qbuf[:, :] = q32.astype(jnp.bfloat16)

        for d in range(1, N_DEV):
            src = lax.rem(my - d + N_DEV, N_DEV)
            pltpu.make_async_remote_copy(
                src_ref=qg_buf.at[:, pl.ds(0, HD_PER)],
                dst_ref=qg_buf.at[:, pl.ds(src * HD_PER, HD_PER)],
                send_sem=qg_send.at[d - 1],
                recv_sem=qg_recv.at[d - 1],
                device_id=(my,),
                device_id_type=pl.DeviceIdType.MESH,
            ).wait_recv()

        for grp in range(N_DEV):
            if grp < N_DEV - 1:
                nxt = start_stage(grp + 1, (grp + 1) % 2)
            pend[0].wait()
            pend[1].wait()
            s0 = (grp % 2) * SKV_PER
            for k in range(HQ_PER):
                g = grp * HQ_PER + k
                c0 = g * DH
                qh = qg_buf[:, c0:c0 + DH]
                kh = kst[s0:s0 + SKV_PER, k * DH:(k + 1) * DH]
                vh = vst[s0:s0 + SKV_PER, k * DH:(k + 1) * DH]
                sT = lax.dot_general(
                    kh, qh, (((1,), (1,)), ((), ())),
                    preferred_element_type=jnp.float32) * SCALE
                m = jnp.max(sT, axis=0, keepdims=True)
                w = jnp.exp(sT - m)
                l = jnp.sum(w, axis=0, keepdims=True)
                cT = lax.dot_general(
                    vh, w.astype(jnp.bfloat16), (((0,), (0,)), ((), ())),
                    preferred_element_type=jnp.float32)
                ctxT[c0:c0 + DH, :] = cT.astype(jnp.bfloat16)
                mlT[g:g + 1, :] = m
                mlT[HQ + g:HQ + g + 1, :] = l
            if grp < N_DEV - 1:
                pend = nxt

        pc_sends = []
        for d in range(1, N_DEV):
            t = lax.rem(my + d, N_DEV)
            rdma = pltpu.make_async_remote_copy(
                src_ref=ctxT.at[pl.ds(t * HD_PER, HD_PER), :],
                dst_ref=gctxT.at[pl.ds((d - 1) * HD_PER, HD_PER), :],
                send_sem=gc_send.at[d - 1],
                recv_sem=gc_recv.at[d - 1],
                device_id=(t,),
                device_id_type=pl.DeviceIdType.MESH,
            )
            rdma.start()
            pc_sends.append(rdma)
            rdma = pltpu.make_async_remote_copy(
                src_ref=mlT,
                dst_ref=gml.at[pl.ds((d - 1) * 2 * HQ, 2 * HQ), :],
                send_sem=gm_send.at[d - 1],
                recv_sem=gm_recv.at[d - 1],
                device_id=(t,),
                device_id_type=pl.DeviceIdType.MESH,
            )
            rdma.start()
            pc_sends.append(rdma)

        def wait_chunk(c):
            rows0, nrows = (0, SKV_PER) if c == 0 else (SKV_PER, CH1_ROWS)
            for (dst, ssem, rsem) in ((kall, k_send, k_recv),
                                      (vall, v_send, v_recv)):
                pltpu.make_async_remote_copy(
                    src_ref=k_any.at[pl.ds(0, nrows), pl.ds(0, HD_PER)],
                    dst_ref=dst.at[pl.ds(rows0, nrows), :],
                    send_sem=ssem.at[0],
                    recv_sem=rsem.at[c],
                    device_id=(my,),
                    device_id_type=pl.DeviceIdType.MESH,
                ).wait_recv()

        def tile_heads(t):
            r0 = N_GLOB + t * LOC_TILE
            w0 = max(0, r0 - LOCAL_W)

            def f(h, carry):
                c0 = h * DH
                ql = qbuf[r0:r0 + LOC_TILE, pl.ds(c0, DH)]
                k_win = kall[w0:w0 + W_WIN, pl.ds(c0, DH)]
                v_win = vall[w0:w0 + W_WIN, pl.ds(c0, DH)]
                sw = lax.dot_general(
                    ql, k_win, (((1,), (1,)), ((), ())),
                    preferred_element_type=jnp.float32) * SCALE
                qi = lax.broadcasted_iota(jnp.int32, (LOC_TILE, W_WIN), 0) + r0
                ki = lax.broadcasted_iota(jnp.int32, (LOC_TILE, W_WIN), 1) + w0
                mask = (jnp.abs(qi - ki) <= LOCAL_W) | (ki < N_GLOB)
                sw = jnp.where(mask, sw, -1e9)
                if t == 0:
                    m = jnp.max(sw, axis=1, keepdims=True)
                    ww = jnp.exp(sw - m)
                    den = jnp.sum(ww, axis=1, keepdims=True)
                    ctx_l = jnp.dot(ww.astype(jnp.bfloat16), v_win,
                                    preferred_element_type=jnp.float32) / den
                else:
                    k_blk = kall[:N_GLOB, pl.ds(c0, DH)]
                    v_blk = vall[:N_GLOB, pl.ds(c0, DH)]
                    sb = lax.dot_general(
                        ql, k_blk, (((1,), (1,)), ((), ())),
                        preferred_element_type=jnp.float32) * SCALE
                    m = jnp.maximum(jnp.max(sw, axis=1, keepdims=True),
                                    jnp.max(sb, axis=1, keepdims=True))
                    ww = jnp.exp(sw - m)
                    wb = jnp.exp(sb - m)
                    den = (jnp.sum(ww, axis=1, keepdims=True)
                           + jnp.sum(wb, axis=1, keepdims=True))
                    ctx_l = (jnp.dot(ww.astype(jnp.bfloat16), v_win,
                                     preferred_element_type=jnp.float32)
                             + jnp.dot(wb.astype(jnp.bfloat16), v_blk,
                                       preferred_element_type=jnp.float32)
                             ) / den
                ctx_buf[r0:r0 + LOC_TILE, pl.ds(c0, DH)] = (
                    ctx_l.astype(jnp.bfloat16))
                return carry

            return f

        wait_chunk(0)
        for t in (0, 1, 2):
            lax.fori_loop(0, HQ_PER, tile_heads(t), 0)
        wait_chunk(1)
        lax.fori_loop(0, HQ_PER, tile_heads(3), 0)

        for d in range(1, N_DEV):
            pltpu.make_async_remote_copy(
                src_ref=ctxT.at[pl.ds(0, HD_PER), :],
                dst_ref=gctxT.at[pl.ds((d - 1) * HD_PER, HD_PER), :],
                send_sem=gc_send.at[d - 1],
                recv_sem=gc_recv.at[d - 1],
                device_id=(my,),
                device_id_type=pl.DeviceIdType.MESH,
            ).wait_recv()
            pltpu.make_async_remote_copy(
                src_ref=mlT,
                dst_ref=gml.at[pl.ds((d - 1) * 2 * HQ, 2 * HQ), :],
                send_sem=gm_send.at[d - 1],
                recv_sem=gm_recv.at[d - 1],
                device_id=(my,),
                device_id_type=pl.DeviceIdType.MESH,
            ).wait_recv()
        for rdma in qg_sends + pc_sends:
            rdma.wait_send()

        m_src = [mlT[pl.ds(8 * my, HQ_PER), :]]
        l_src = [mlT[pl.ds(8 * (my + 4), HQ_PER), :]]
        for d in range(1, N_DEV):
            m_src.append(gml[pl.ds(8 * ((d - 1) * 8 + my), HQ_PER), :])
            l_src.append(gml[pl.ds(8 * ((d - 1) * 8 + my + 4), HQ_PER), :])
        ident = jnp.eye(N_GLOB, dtype=jnp.bfloat16)
        for j in range(HQ_PER):
            ms = [mm[j:j + 1, :] for mm in m_src]
            ls = [ll[j:j + 1, :] for ll in l_src]
            M = ms[0]
            for mm in ms[1:]:
                M = jnp.maximum(M, mm)
            scs = [jnp.exp(mm - M) for mm in ms]
            L = ls[0] * scs[0]
            for ll, sc in zip(ls[1:], scs[1:]):
                L = L + ll * sc
            ct = (ctxT[pl.ds(8 * (128 * my + 16 * j), DH), :]
                  .astype(jnp.float32) * scs[0])
            for d in range(1, N_DEV):
                r = (d - 1) * HD_PER + j * DH
                ct = ct + (gctxT[r:r + DH, :].astype(jnp.float32)
                           * scs[d])
            ct = ct / L
            ctx32 = lax.dot_general(
                ident, ct.astype(jnp.bfloat16), (((1,), (1,)), ((), ())),
                preferred_element_type=jnp.float32)
            ctx_buf[:N_GLOB, j * DH:(j + 1) * DH] = ctx32.astype(jnp.bfloat16)

        @pl.when(my == 0)
        def _():
            for t in range(N_DEV):
                for (src, dst, ssem, rsem) in (
                    (k_any, kall, k_send, k_recv),
                    (v_any, vall, v_send, v_recv),
                ):
                    pltpu.make_async_remote_copy(
                        src_ref=src.at[:, pl.ds(t * HD_PER, HD_PER)],
                        dst_ref=dst.at[pl.ds(0, SKV_PER), :],
                        send_sem=ssem.at[t],
                        recv_sem=rsem.at[0],
                        device_id=(t,),
                        device_id_type=pl.DeviceIdType.MESH,
                    ).wait_send()

        @pl.when(my == 1)
        def _():
            for t in range(N_DEV):
                for (src, dst, ssem, rsem) in (
                    (k_any, kall, k_send, k_recv),
                    (v_any, vall, v_send, v_recv),
                ):
                    pltpu.make_async_remote_copy(
                        src_ref=src.at[pl.ds(0, CH1_ROWS),
                                       pl.ds(t * HD_PER, HD_PER)],
                        dst_ref=dst.at[pl.ds(SKV_PER, CH1_ROWS), :],
                        send_sem=ssem.at[t],
                        recv_sem=rsem.at[1],
                        device_id=(t,),
                        device_id_type=pl.DeviceIdType.MESH,
                    ).wait_send()

        partial = jnp.dot(ctx_buf[:, :], wo_ref[:, :],
                          preferred_element_type=jnp.float32)

        ctx_buf[:, :] = partial.astype(jnp.bfloat16)
        out_ref[:, :] = partial
        rs_rdmas = []
        for d in range(1, N_DEV):
            t = lax.rem(my + d, N_DEV)
            rdma = pltpu.make_async_remote_copy(
                src_ref=ctx_buf.at[pl.ds(t * QROWS, QROWS), :],
                dst_ref=rs_buf.at[d - 1],
                send_sem=rs_send.at[d - 1],
                recv_sem=rs_recv.at[d - 1],
                device_id=(t,),
                device_id_type=pl.DeviceIdType.MESH,
            )
            rdma.start()
            rs_rdmas.append(rdma)
        for d in range(1, N_DEV):
            src = lax.rem(my - d + N_DEV, N_DEV)
            pltpu.make_async_remote_copy(
                src_ref=ctx_buf.at[pl.ds(0, QROWS), :],
                dst_ref=rs_buf.at[d - 1],
                send_sem=rs_send.at[d - 1],
                recv_sem=rs_recv.at[d - 1],
                device_id=(src,),
                device_id_type=pl.DeviceIdType.MESH,
            ).wait_recv()
        for rdma in rs_rdmas:
            rdma.wait_send()

        red = out_ref[pl.ds(my * QROWS, QROWS), :]
        for j in range(N_DEV - 1):
            red = red + rs_buf[j, :, :].astype(jnp.float32)
        qbuf[pl.ds(my * QROWS, QROWS), :] = red.astype(jnp.bfloat16)

        ag_rdmas = []
        for d in range(1, N_DEV):
            t = lax.rem(my + d, N_DEV)
            rdma = pltpu.make_async_remote_copy(
                src_ref=qbuf.at[pl.ds(my * QROWS, QROWS), :],
                dst_ref=qbuf.at[pl.ds(my * QROWS, QROWS), :],
                send_sem=ag_send.at[d - 1],
                recv_sem=ag_recv.at[d - 1],
                device_id=(t,),
                device_id_type=pl.DeviceIdType.MESH,
            )
            rdma.start()
            ag_rdmas.append(rdma)
        for d in range(1, N_DEV):
            src = lax.rem(my - d + N_DEV, N_DEV)
            pltpu.make_async_remote_copy(
                src_ref=qbuf.at[pl.ds(0, QROWS), :],
                dst_ref=qbuf.at[pl.ds(src * QROWS, QROWS), :],
                send_sem=ag_send.at[d - 1],
                recv_sem=ag_recv.at[d - 1],
                device_id=(src,),
                device_id_type=pl.DeviceIdType.MESH,
            ).wait_recv()
        for rdma in ag_rdmas:
            rdma.wait_send()

        out_ref[:, :] = qbuf[:, :].astype(jnp.float32)

    out = pl.pallas_call(
        body,
        out_shape=jax.ShapeDtypeStruct((SQ, DM), jnp.float32),
        in_specs=[
            pl.BlockSpec(memory_space=pltpu.VMEM),
            pl.BlockSpec(memory_space=pltpu.VMEM),
            pl.BlockSpec(memory_space=pltpu.MemorySpace.HBM),
            pl.BlockSpec(memory_space=pltpu.MemorySpace.HBM),
            pl.BlockSpec(memory_space=pltpu.VMEM),
        ],
        out_specs=pl.BlockSpec(memory_space=pltpu.VMEM),
        scratch_shapes=[
            pltpu.VMEM((KV_EXT, HD_PER), jnp.bfloat16),
            pltpu.VMEM((KV_EXT, HD_PER), jnp.bfloat16),
            pltpu.VMEM((2 * SKV_PER, HD_PER), jnp.bfloat16),
            pltpu.VMEM((2 * SKV_PER, HD_PER), jnp.bfloat16),
            pltpu.VMEM((SQ, HD_PER), jnp.bfloat16),
            pltpu.VMEM((SQ, HD_PER), jnp.bfloat16),
            pltpu.VMEM((N_GLOB, HD_ALL), jnp.bfloat16),
            pltpu.VMEM((HD_ALL, N_GLOB), jnp.bfloat16),
            pltpu.VMEM((2 * HQ, N_GLOB), jnp.float32),
            pltpu.VMEM(((N_DEV - 1) * HD_PER, N_GLOB), jnp.bfloat16),
            pltpu.VMEM(((N_DEV - 1) * 2 * HQ, N_GLOB), jnp.float32),
            pltpu.VMEM((N_DEV - 1, QROWS, DM), jnp.bfloat16),
            pltpu.SemaphoreType.DMA((N_DEV,)),
            pltpu.SemaphoreType.DMA((N_DEV,)),
            pltpu.SemaphoreType.DMA((2,)),
            pltpu.SemaphoreType.DMA((2,)),
            pltpu.SemaphoreType.DMA((N_DEV - 1,)),
            pltpu.SemaphoreType.DMA((N_DEV - 1,)),
            pltpu.SemaphoreType.DMA((N_DEV - 1,)),
            pltpu.SemaphoreType.DMA((N_DEV - 1,)),
            pltpu.SemaphoreType.DMA((N_DEV - 1,)),
            pltpu.SemaphoreType.DMA((N_DEV - 1,)),
            pltpu.SemaphoreType.DMA((N_DEV - 1,)),
            pltpu.SemaphoreType.DMA((N_DEV - 1,)),
            pltpu.SemaphoreType.DMA((N_DEV - 1,)),
            pltpu.SemaphoreType.DMA((N_DEV - 1,)),
            pltpu.SemaphoreType.DMA((4,)),
        ],
        compiler_params=pltpu.CompilerParams(collective_id=0),
    )(xb, Wqb, K2, V2, Wob)
    return out.reshape(1, SQ, DM)


# device time: 188620 ns/iter; 1.0309x vs baseline; 1.0309x over previous
import jax
import jax.numpy as jnp
from jax import lax
from jax.experimental import pallas as pl
from jax.experimental.pallas import tpu as pltpu

N_DEV = 4
SQ = 1024
SKV_PER = 1024
SKV = N_DEV * SKV_PER
HQ_PER = 8
DH = 128
DM = 1024
HD_PER = HQ_PER * DH
SCALE = 0.08838834764831843
LOCAL_W = 128
N_GLOB = 32
KV_LOC = 2 * SKV_PER
LOC_TILE = 248
W_WIN = 512
QROWS = SQ // N_DEV


def kernel(x, Wq, K_ext, V_ext, Wo):
    xb = x.reshape(SQ, DM).astype(jnp.bfloat16)
    Wqb = Wq.astype(jnp.bfloat16)
    Wob = Wo.astype(jnp.bfloat16)
    K2 = K_ext.reshape(SKV_PER, 32 * DH).astype(jnp.bfloat16)
    V2 = V_ext.reshape(SKV_PER, 32 * DH).astype(jnp.bfloat16)

    def body(x_ref, wq_ref, k_any, v_any, wo_ref, out_ref,
             kall, vall, qbuf, ctx_buf, pbuf, rs_buf, ag_buf,
             k_send, v_send, k_recv, v_recv,
             rs_send, rs_recv, ag_send, ag_recv):
        my = lax.axis_index("i")

        barrier_sem = pltpu.get_barrier_semaphore()
        for d in range(1, N_DEV):
            peer = lax.rem(my + d, N_DEV)
            pl.semaphore_signal(
                barrier_sem, inc=1,
                device_id=(peer,), device_id_type=pl.DeviceIdType.MESH,
            )
        pl.semaphore_wait(barrier_sem, N_DEV - 1)

        sends = []
        for d in range(N_DEV):
            t = lax.rem(my + d, N_DEV)
            for (src_ref, dst, ssem, rsem) in (
                (k_any, kall, k_send, k_recv),
                (v_any, vall, v_send, v_recv),
            ):
                rdma = pltpu.make_async_remote_copy(
                    src_ref=src_ref.at[:, pl.ds(t * HD_PER, HD_PER)],
                    dst_ref=dst.at[pl.ds(my * SKV_PER, SKV_PER), :],
                    send_sem=ssem.at[d],
                    recv_sem=rsem.at[d],
                    device_id=(t,),
                    device_id_type=pl.DeviceIdType.MESH,
                )
                rdma.start()
                sends.append(rdma)

        q32 = jnp.dot(x_ref[:, :], wq_ref[:, :], preferred_element_type=jnp.float32)
        qbuf[:, :] = q32.astype(jnp.bfloat16)

        def wait_chunk(c):
            dc = lax.rem(my - c + N_DEV, N_DEV)
            for (dst, ssem, rsem) in ((kall, k_send, k_recv),
                                      (vall, v_send, v_recv)):
                recv = pltpu.make_async_remote_copy(
                    src_ref=k_any.at[:, pl.ds(0, HD_PER)],
                    dst_ref=dst.at[pl.ds(c * SKV_PER, SKV_PER), :],
                    send_sem=ssem.at[dc],
                    recv_sem=rsem.at[dc],
                    device_id=(my,),
                    device_id_type=pl.DeviceIdType.MESH,
                )
                recv.wait_recv()

        def tile_heads(t):
            r0 = N_GLOB + t * LOC_TILE
            w0 = max(0, r0 - LOCAL_W)

            def f(h, carry):
                c0 = h * DH
                ql = qbuf[r0:r0 + LOC_TILE, pl.ds(c0, DH)]
                k_win = kall[w0:w0 + W_WIN, pl.ds(c0, DH)]
                v_win = vall[w0:w0 + W_WIN, pl.ds(c0, DH)]
                sw = lax.dot_general(
                    ql, k_win, (((1,), (1,)), ((), ())),
                    preferred_element_type=jnp.float32) * SCALE
                qi = lax.broadcasted_iota(jnp.int32, (LOC_TILE, W_WIN), 0) + r0
                ki = lax.broadcasted_iota(jnp.int32, (LOC_TILE, W_WIN), 1) + w0
                mask = (jnp.abs(qi - ki) <= LOCAL_W) | (ki < N_GLOB)
                sw = jnp.where(mask, sw, -1e9)
                if t == 0:
                    m = jnp.max(sw, axis=1, keepdims=True)
                    ww = jnp.exp(sw - m)
                    den = jnp.sum(ww, axis=1, keepdims=True)
                    ctx_l = jnp.dot(ww.astype(jnp.bfloat16), v_win,
                                    preferred_element_type=jnp.float32) / den
                else:
                    k_blk = kall[:N_GLOB, pl.ds(c0, DH)]
                    v_blk = vall[:N_GLOB, pl.ds(c0, DH)]
                    sb = lax.dot_general(
                        ql, k_blk, (((1,), (1,)), ((), ())),
                        preferred_element_type=jnp.float32) * SCALE
                    m = jnp.maximum(jnp.max(sw, axis=1, keepdims=True),
                                    jnp.max(sb, axis=1, keepdims=True))
                    ww = jnp.exp(sw - m)
                    wb = jnp.exp(sb - m)
                    den = (jnp.sum(ww, axis=1, keepdims=True)
                           + jnp.sum(wb, axis=1, keepdims=True))
                    ctx_l = (jnp.dot(ww.astype(jnp.bfloat16), v_win,
                                     preferred_element_type=jnp.float32)
                             + jnp.dot(wb.astype(jnp.bfloat16), v_blk,
                                       preferred_element_type=jnp.float32)
                             ) / den
                ctx_buf[r0:r0 + LOC_TILE, pl.ds(c0, DH)] = (
                    ctx_l.astype(jnp.bfloat16))
                return carry

            return f

        wait_chunk(0)
        for t in (0, 1, 2):
            lax.fori_loop(0, HQ_PER, tile_heads(t), 0)
        wait_chunk(1)
        lax.fori_loop(0, HQ_PER, tile_heads(3), 0)
        wait_chunk(2)
        wait_chunk(3)
        for rdma in sends:
            rdma.wait_send()

        def global_heads(h, carry):
            c0 = h * DH
            qg = qbuf[:N_GLOB, pl.ds(c0, DH)]
            kh = kall[:, pl.ds(c0, DH)]
            vh = vall[:, pl.ds(c0, DH)]
            sg = lax.dot_general(
                qg, kh, (((1,), (1,)), ((), ())),
                preferred_element_type=jnp.float32) * SCALE
            sg = sg - jnp.max(sg, axis=1, keepdims=True)
            wg = jnp.exp(sg)
            den_g = jnp.sum(wg, axis=1, keepdims=True)
            ctx_g = jnp.dot(wg.astype(jnp.bfloat16), vh,
                            preferred_element_type=jnp.float32) / den_g
            ctx_buf[:N_GLOB, pl.ds(c0, DH)] = ctx_g.astype(jnp.bfloat16)
            return carry

        lax.fori_loop(0, HQ_PER, global_heads, 0)

        partial = jnp.dot(ctx_buf[:, :], wo_ref[:, :],
                          preferred_element_type=jnp.float32)

        pbuf[:, :] = partial.astype(jnp.bfloat16)
        out_ref[:, :] = partial
        rs_rdmas = []
        for d in range(1, N_DEV):
            t = lax.rem(my + d, N_DEV)
            rdma = pltpu.make_async_remote_copy(
                src_ref=pbuf.at[pl.ds(t * QROWS, QROWS), :],
                dst_ref=rs_buf.at[d - 1],
                send_sem=rs_send.at[d - 1],
                recv_sem=rs_recv.at[d - 1],
                device_id=(t,),
                device_id_type=pl.DeviceIdType.MESH,
            )
            rdma.start()
            rs_rdmas.append(rdma)
        for d in range(1, N_DEV):
            src = lax.rem(my - d + N_DEV, N_DEV)
            recv = pltpu.make_async_remote_copy(
                src_ref=pbuf.at[pl.ds(0, QROWS), :],
                dst_ref=rs_buf.at[d - 1],
                send_sem=rs_send.at[d - 1],
                recv_sem=rs_recv.at[d - 1],
                device_id=(src,),
                device_id_type=pl.DeviceIdType.MESH,
            )
            recv.wait_recv()
        for rdma in rs_rdmas:
            rdma.wait_send()

        red = out_ref[pl.ds(my * QROWS, QROWS), :]
        for j in range(N_DEV - 1):
            red = red + rs_buf[j, :, :].astype(jnp.float32)
        ag_buf[pl.ds(my * QROWS, QROWS), :] = red.astype(jnp.bfloat16)

        ag_rdmas = []
        for d in range(1, N_DEV):
            t = lax.rem(my + d, N_DEV)
            rdma = pltpu.make_async_remote_copy(
                src_ref=ag_buf.at[pl.ds(my * QROWS, QROWS), :],
                dst_ref=ag_buf.at[pl.ds(my * QROWS, QROWS), :],
                send_sem=ag_send.at[d - 1],
                recv_sem=ag_recv.at[d - 1],
                device_id=(t,),
                device_id_type=pl.DeviceIdType.MESH,
            )
            rdma.start()
            ag_rdmas.append(rdma)
        for d in range(1, N_DEV):
            src = lax.rem(my - d + N_DEV, N_DEV)
            recv = pltpu.make_async_remote_copy(
                src_ref=ag_buf.at[pl.ds(0, QROWS), :],
                dst_ref=ag_buf.at[pl.ds(src * QROWS, QROWS), :],
                send_sem=ag_send.at[d - 1],
                recv_sem=ag_recv.at[d - 1],
                device_id=(src,),
                device_id_type=pl.DeviceIdType.MESH,
            )
            recv.wait_recv()
        for rdma in ag_rdmas:
            rdma.wait_send()

        out_ref[:, :] = ag_buf[:, :].astype(jnp.float32)

    out = pl.pallas_call(
        body,
        out_shape=jax.ShapeDtypeStruct((SQ, DM), jnp.float32),
        in_specs=[
            pl.BlockSpec(memory_space=pltpu.VMEM),
            pl.BlockSpec(memory_space=pltpu.VMEM),
            pl.BlockSpec(memory_space=pltpu.MemorySpace.HBM),
            pl.BlockSpec(memory_space=pltpu.MemorySpace.HBM),
            pl.BlockSpec(memory_space=pltpu.VMEM),
        ],
        out_specs=pl.BlockSpec(memory_space=pltpu.VMEM),
        scratch_shapes=[
            pltpu.VMEM((SKV, HD_PER), jnp.bfloat16),
            pltpu.VMEM((SKV, HD_PER), jnp.bfloat16),
            pltpu.VMEM((SQ, HD_PER), jnp.bfloat16),
            pltpu.VMEM((SQ, HD_PER), jnp.bfloat16),
            pltpu.VMEM((SQ, DM), jnp.bfloat16),
            pltpu.VMEM((N_DEV - 1, QROWS, DM), jnp.bfloat16),
            pltpu.VMEM((SQ, DM), jnp.bfloat16),
            pltpu.SemaphoreType.DMA((N_DEV,)),
            pltpu.SemaphoreType.DMA((N_DEV,)),
            pltpu.SemaphoreType.DMA((N_DEV,)),
            pltpu.SemaphoreType.DMA((N_DEV,)),
            pltpu.SemaphoreType.DMA((N_DEV - 1,)),
            pltpu.SemaphoreType.DMA((N_DEV - 1,)),
            pltpu.SemaphoreType.DMA((N_DEV - 1,)),
            pltpu.SemaphoreType.DMA((N_DEV - 1,)),
        ],
        compiler_params=pltpu.CompilerParams(collective_id=0),
    )(xb, Wqb, K2, V2, Wob)
    return out.reshape(1, SQ, DM)
